# Initial kernel scaffold; baseline (speedup 1.0000x reference)
import jax
import jax.numpy as jnp
from jax.experimental import pallas as pl


def kernel(input):
    raise NotImplementedError("write your pallas kernel here")


# speedup vs baseline: 18.3376x; 18.3376x over previous
import jax
import jax.numpy as jnp
from jax.experimental import pallas as pl
from jax.experimental.pallas import tpu as pltpu

_ROWS_PER_BLOCK = 256
_BISECT_ITERS = 30


def _sparsemax_block(x_ref, o_ref):
    x = x_ref[...]
    m = jnp.max(x, axis=1, keepdims=True)
    lo = m - 1.0
    hi = m
    for _ in range(_BISECT_ITERS):
        mid = 0.5 * (lo + hi)
        s = jnp.sum(jnp.maximum(x - mid, 0.0), axis=1, keepdims=True)
        ge = s >= 1.0
        lo = jnp.where(ge, mid, lo)
        hi = jnp.where(ge, hi, mid)
    mask = x > lo
    cnt = jnp.sum(mask.astype(x.dtype), axis=1, keepdims=True)
    ssum = jnp.sum(jnp.where(mask, x, 0.0), axis=1, keepdims=True)
    tau = (ssum - 1.0) / jnp.maximum(cnt, 1.0)
    o_ref[...] = jnp.maximum(x - tau, 0.0)


def kernel(input):
    b, s, d = input.shape
    n = b * s
    x2 = input.reshape(n, d)
    out = pl.pallas_call(
        _sparsemax_block,
        grid=(n // _ROWS_PER_BLOCK,),
        in_specs=[pl.BlockSpec((_ROWS_PER_BLOCK, d), lambda i: (i, 0))],
        out_specs=pl.BlockSpec((_ROWS_PER_BLOCK, d), lambda i: (i, 0)),
        out_shape=jax.ShapeDtypeStruct((n, d), input.dtype),
        compiler_params=pltpu.CompilerParams(
            dimension_semantics=("arbitrary",),
        ),
    )(x2)
    return out.reshape(b, s, d)
